# baseline (speedup 1.0000x reference)
import functools

import jax
import jax.numpy as jnp
from jax import lax
from jax.experimental import pallas as pl
from jax.experimental.pallas import tpu as pltpu
from jax.experimental.pallas import tpu_sc as plsc

EMBED = 64
NC = 2
NS = 16
NW = NC * NS

SUB = 128
IDXROWS = 8
GROUP = IDXROWS * SUB
CHUNK = 256
NSUB = CHUNK // SUB
QPG = GROUP // CHUNK
LANES = 16


def _make_sc_gather(B):
    PW = B // NW
    G = PW // GROUP

    mesh = plsc.VectorSubcoreMesh(core_axis_name="c", subcore_axis_name="s")

    @functools.partial(
        pl.kernel,
        mesh=mesh,
        out_type=jax.ShapeDtypeStruct((B // 200, 200, EMBED), jnp.float32),
        scratch_types=[
            pltpu.VMEM((IDXROWS, SUB), jnp.int32),
            pltpu.VMEM((CHUNK, 2 * EMBED), jnp.float32),
            pltpu.VMEM((CHUNK, EMBED), jnp.float32),
            pltpu.SemaphoreType.DMA,
        ],
    )
    def k(idx_hbm, table_hbm, out3_hbm, idx_v, rows_v, rows_t, sem):
        out_hbm = out3_hbm.reshape(B, EMBED)
        wid = lax.axis_index("s") * NC + lax.axis_index("c")

        def body(g, carry):
            pltpu.sync_copy(idx_hbm.at[wid, g], idx_v)
            for q in range(QPG):
                base = wid * PW + g * GROUP + q * CHUNK
                for j in range(NSUB):
                    pltpu.async_copy(
                        table_hbm.at[idx_v.at[q * NSUB + j]],
                        rows_v.at[pl.ds(j * SUB, SUB)],
                        sem,
                    ).wait()

                def repack(r, c2):
                    for c in range(EMBED // LANES):
                        rows_t[r, pl.ds(c * LANES, LANES)] = (
                            rows_v[r, pl.ds(c * LANES, LANES)])
                    return c2

                lax.fori_loop(0, CHUNK, repack, 0)
                pltpu.sync_copy(rows_t, out_hbm.at[pl.ds(base, CHUNK)])
            return carry

        lax.fori_loop(0, G, body, 0)

    return k


def kernel(positions, table):
    batch, seq = positions.shape
    B = batch * seq
    idx = positions.reshape(NW, B // (NW * GROUP), IDXROWS, SUB).astype(jnp.int32)
    table_pad = jnp.pad(table, ((0, 0), (0, EMBED)))
    out = _make_sc_gather(B)(idx, table_pad)
    return out.reshape(batch, seq, EMBED) if out.shape != (batch, seq, EMBED) else out


# speedup vs baseline: 1.0909x; 1.0909x over previous
import functools

import jax
import jax.numpy as jnp
from jax import lax
from jax.experimental import pallas as pl
from jax.experimental.pallas import tpu as pltpu
from jax.experimental.pallas import tpu_sc as plsc

EMBED = 64
NC = 2
NS = 16
NW = NC * NS

ROWS_PER_CHUNK = 2
NSUB = 4
NBUF = 2


def _make_sc_gather(batch, seq):
    B = batch * seq
    PW = B // NW
    BR = batch // NW
    G = BR // ROWS_PER_CHUNK
    CHUNK = ROWS_PER_CHUNK * seq
    SUB = CHUNK // NSUB

    mesh = plsc.VectorSubcoreMesh(core_axis_name="c", subcore_axis_name="s")

    @functools.partial(
        pl.kernel,
        mesh=mesh,
        out_type=jax.ShapeDtypeStruct((batch, seq, EMBED), jnp.float32),
        scratch_types=[
            pltpu.VMEM((NSUB, SUB), jnp.int32),
            pltpu.VMEM((NSUB, SUB), jnp.int32),
            pltpu.VMEM((ROWS_PER_CHUNK, seq, EMBED), jnp.float32),
            pltpu.VMEM((ROWS_PER_CHUNK, seq, EMBED), jnp.float32),
            pltpu.SemaphoreType.DMA,
            pltpu.SemaphoreType.DMA,
            pltpu.SemaphoreType.DMA,
            pltpu.SemaphoreType.DMA,
            pltpu.SemaphoreType.DMA,
            pltpu.SemaphoreType.DMA,
        ],
        compiler_params=pltpu.CompilerParams(use_tc_tiling_on_sc=False),
    )
    def k(idx_hbm, table_hbm, out_hbm,
          idx_v0, idx_v1, rows_v0, rows_v1,
          si0, si1, sg0, sg1, so0, so1):
        wid = lax.axis_index("s") * NC + lax.axis_index("c")
        idx_v = (idx_v0, idx_v1)
        rows_v = (rows_v0, rows_v1)
        sem_i = (si0, si1)
        sem_g = (sg0, sg1)
        sem_o = (so0, so1)

        def dst(b, j):
            r = (j * SUB) // seq
            s = (j * SUB) % seq
            return rows_v[b].at[r, pl.ds(s, SUB)]

        for b in range(NBUF):
            pltpu.async_copy(idx_hbm.at[wid, b], idx_v[b], sem_i[b])

        def body(c2, carry):
            for b in range(NBUF):
                c = NBUF * c2 + b
                brow = wid * BR + c * ROWS_PER_CHUNK
                pltpu.make_async_copy(idx_hbm.at[wid, c], idx_v[b], sem_i[b]).wait()

                @pl.when(c2 >= 1)
                def _():
                    pltpu.make_async_copy(
                        rows_v[b],
                        out_hbm.at[pl.ds(brow, ROWS_PER_CHUNK)],
                        sem_o[b],
                    ).wait()

                for j in range(NSUB):
                    pltpu.async_copy(
                        table_hbm.at[idx_v[b].at[j]], dst(b, j), sem_g[b])
                for j in range(NSUB):
                    pltpu.make_async_copy(
                        table_hbm.at[idx_v[b].at[j]], dst(b, j), sem_g[b]).wait()

                @pl.when(c + NBUF < G)
                def _():
                    pltpu.async_copy(idx_hbm.at[wid, c + NBUF], idx_v[b], sem_i[b])

                pltpu.async_copy(
                    rows_v[b], out_hbm.at[pl.ds(brow, ROWS_PER_CHUNK)], sem_o[b])
            return carry

        lax.fori_loop(0, G // NBUF, body, 0)

        for b in range(NBUF):
            c = G - NBUF + b
            brow = wid * BR + c * ROWS_PER_CHUNK
            pltpu.make_async_copy(
                rows_v[b], out_hbm.at[pl.ds(brow, ROWS_PER_CHUNK)], sem_o[b]
            ).wait()

    return k


def kernel(positions, table):
    batch, seq = positions.shape
    B = batch * seq
    chunk = ROWS_PER_CHUNK * seq
    idx = positions.reshape(NW, B // (NW * chunk), NSUB, chunk // NSUB)
    idx = idx.astype(jnp.int32)
    return _make_sc_gather(batch, seq)(idx, table)


# speedup vs baseline: 1.3936x; 1.2775x over previous
import functools

import jax
import jax.numpy as jnp
from jax import lax
from jax.experimental import pallas as pl
from jax.experimental.pallas import tpu as pltpu
from jax.experimental.pallas import tpu_sc as plsc

EMBED = 64
NC = 2
NS = 16
NW = NC * NS

CHUNK = 128
LANES = 16
NBUF = 2


def _make_sc_gather(B):
    PW = B // NW
    G = PW // CHUNK

    mesh = plsc.VectorSubcoreMesh(core_axis_name="c", subcore_axis_name="s")

    @functools.partial(
        pl.kernel,
        mesh=mesh,
        out_type=jax.ShapeDtypeStruct((B, EMBED), jnp.float32),
        scratch_types=[
            pltpu.VMEM((1, CHUNK), jnp.int32),
            pltpu.VMEM((1, CHUNK), jnp.int32),
            pltpu.VMEM((CHUNK, 2 * EMBED), jnp.float32),
            pltpu.VMEM((CHUNK, 2 * EMBED), jnp.float32),
            pltpu.VMEM((CHUNK, EMBED), jnp.float32),
            pltpu.VMEM((CHUNK, EMBED), jnp.float32),
            pltpu.SemaphoreType.DMA,
            pltpu.SemaphoreType.DMA,
            pltpu.SemaphoreType.DMA,
            pltpu.SemaphoreType.DMA,
            pltpu.SemaphoreType.DMA,
            pltpu.SemaphoreType.DMA,
        ],
    )
    def k(idx_hbm, table_hbm, out_hbm,
          idx_v0, idx_v1, rows_v0, rows_v1, rt_v0, rt_v1,
          si0, si1, sg0, sg1, so0, so1):
        wid = lax.axis_index("s") * NC + lax.axis_index("c")
        idx_v = (idx_v0, idx_v1)
        rows_v = (rows_v0, rows_v1)
        rt_v = (rt_v0, rt_v1)
        sem_i = (si0, si1)
        sem_g = (sg0, sg1)
        sem_o = (so0, so1)

        def fire_gather(slot, c):
            pltpu.async_copy(
                table_hbm.at[idx_v[slot].at[0]], rows_v[slot], sem_g[slot])

        def wait_gather(slot):
            pltpu.make_async_copy(
                table_hbm.at[idx_v[slot].at[0]], rows_v[slot], sem_g[slot]
            ).wait()

        def repack_and_flush(slot, p):
            def repack(r4, carry):
                for dr in range(4):
                    r = r4 * 4 + dr
                    for cc in range(EMBED // LANES):
                        rt_v[slot][r, pl.ds(cc * LANES, LANES)] = (
                            rows_v[slot][r, pl.ds(cc * LANES, LANES)])
                return carry

            lax.fori_loop(0, CHUNK // 4, repack, 0)
            base = wid * PW + p * CHUNK
            pltpu.async_copy(
                rt_v[slot], out_hbm.at[pl.ds(base, CHUNK)], sem_o[slot])

        def wait_flush(slot):
            base = wid * PW
            pltpu.make_async_copy(
                rt_v[slot], out_hbm.at[pl.ds(base, CHUNK)], sem_o[slot]
            ).wait()

        for b in range(NBUF):
            pltpu.async_copy(idx_hbm.at[wid, b], idx_v[b], sem_i[b])

        def body(c2, carry):
            for b in range(NBUF):
                c = NBUF * c2 + b
                q = 1 - b
                pltpu.make_async_copy(
                    idx_hbm.at[wid, c], idx_v[b], sem_i[b]).wait()
                fire_gather(b, c)

                def handle_prev():
                    wait_gather(q)
                    repack_and_flush(q, c - 1)

                    @pl.when(c + 1 < G)
                    def _():
                        pltpu.async_copy(
                            idx_hbm.at[wid, c + 1], idx_v[q], sem_i[q])

                if b == 0:
                    @pl.when(c2 >= 1)
                    def _():
                        @pl.when(c2 >= 2)
                        def _():
                            wait_flush(q)
                        handle_prev()
                else:
                    @pl.when(c2 >= 1)
                    def _():
                        wait_flush(q)
                    handle_prev()
            return carry

        lax.fori_loop(0, G // NBUF, body, 0)

        qf = (G - 1) % NBUF
        wait_flush(qf)
        wait_gather(qf)
        repack_and_flush(qf, G - 1)
        wait_flush(1 - qf)
        wait_flush(qf)

    return k


def kernel(positions, table):
    batch, seq = positions.shape
    B = batch * seq
    idx = positions.reshape(NW, B // (NW * CHUNK), 1, CHUNK).astype(jnp.int32)
    table_pad = jnp.pad(table, ((0, 0), (0, EMBED)))
    out = _make_sc_gather(B)(idx, table_pad)
    return out.reshape(batch, seq, EMBED)
